# baseline (speedup 1.0000x reference)
import functools

import jax
import jax.numpy as jnp
from jax import lax
from jax.experimental import pallas as pl
from jax.experimental.pallas import tpu as pltpu
from jax.experimental.pallas import tpu_sc as plsc

BATCH = 16384
EMBED = 32
SEQ = 20
L = 16

NC = 2
NS = 16
NW = NC * NS
BPW = BATCH // NW
CHUNK = 128
NCHUNK = BPW // CHUNK
TOK_PER_CHUNK = CHUNK * SEQ
IDX_DMA = 128

_mesh = plsc.VectorSubcoreMesh(core_axis_name="c", subcore_axis_name="s")

_IL = plsc.PackFormat.INTERLEAVED


@functools.partial(
    pl.kernel,
    out_type=jax.ShapeDtypeStruct((BATCH, 2 * EMBED), jnp.float32),
    mesh=_mesh,
    compiler_params=pltpu.CompilerParams(needs_layout_passes=False,
                                         use_tc_tiling_on_sc=False),
    scratch_types=(
        [pltpu.VMEM((BPW,), jnp.int32),
         pltpu.VMEM((BPW, SEQ), jnp.int32),
         pltpu.VMEM((BPW,), jnp.float32),
         pltpu.VMEM((BPW,), jnp.float32),
         pltpu.VMEM((1, EMBED), jnp.bfloat16)]
        + [pltpu.VMEM((TOK_PER_CHUNK,), jnp.int32)] * 2
        + [pltpu.VMEM((CHUNK, EMBED), jnp.bfloat16)] * 2
        + [pltpu.VMEM((TOK_PER_CHUNK, EMBED), jnp.bfloat16)] * 2
        + [pltpu.VMEM((CHUNK, 2 * EMBED), jnp.float32)] * 2
        + [pltpu.SemaphoreType.DMA] * 4
    ),
)
def _sc_kernel(title_hbm, tok2d_hbm, ttab_hbm, xtab_hbm, out_hbm,
               tidx_v, tokidx_v, n0_v, scale_v, row0_v,
               flat0, flat1, trows0, trows1, tokbuf0, tokbuf1,
               rowbuf0, rowbuf1, sg0, sg1, so0, so1):
    wid = lax.axis_index("s") * NC + lax.axis_index("c")
    base = wid * BPW
    flat = (flat0, flat1)
    trows = (trows0, trows1)
    tokbuf = (tokbuf0, tokbuf1)
    rowbuf = (rowbuf0, rowbuf1)
    sem_g = (sg0, sg1)
    sem_o = (so0, so1)

    pltpu.sync_copy(title_hbm.at[pl.ds(base, BPW)], tidx_v)
    pltpu.sync_copy(tok2d_hbm.at[pl.ds(base, BPW)], tokidx_v)
    pltpu.sync_copy(xtab_hbm.at[pl.ds(0, 1)], row0_v)

    iota = lax.iota(jnp.int32, L)
    seqv = jnp.full((L,), SEQ, jnp.int32)

    def fire(c):
        p = c & 1

        def build_body(k, _):
            pg = c * TOK_PER_CHUNK + k * L + iota
            flat[p][pl.ds(k * L, L)] = plsc.load_gather(
                tokidx_v, [pg // seqv, pg % seqv])
            return 0

        lax.fori_loop(0, TOK_PER_CHUNK // L, build_body, 0)

        descs = [pltpu.async_copy(
            ttab_hbm.at[tidx_v.at[pl.ds(c * CHUNK, CHUNK)]],
            trows[p], sem_g[p])]
        for j in range(TOK_PER_CHUNK // IDX_DMA):
            descs.append(pltpu.async_copy(
                xtab_hbm.at[flat[p].at[pl.ds(j * IDX_DMA, IDX_DMA)]],
                tokbuf[p].at[pl.ds(j * IDX_DMA, IDX_DMA)], sem_g[p]))
        return descs

    descs = {0: fire(0)}

    one = jnp.ones((L,), jnp.float32)
    zero = jnp.zeros((L,), jnp.float32)
    full = jnp.full((L,), float(SEQ), jnp.float32)

    def count_body(g, _):
        sidx = iota + g * L
        zc = zero
        for t in range(SEQ):
            ids = plsc.load_gather(tokidx_v, [sidx, jnp.full((L,), t,
                                                             jnp.int32)])
            zc = zc + jnp.where(ids == 0, one, zero)
        n0_v[pl.ds(g * L, L)] = zc
        scale_v[pl.ds(g * L, L)] = one / jnp.maximum(full - zc, one)
        return 0

    lax.fori_loop(0, BPW // L, count_body, 0)

    r0e, r0o = plsc.unpack(row0_v[0, :], format=_IL)
    iota2 = iota * 2
    zero_i = jnp.zeros((L,), jnp.int32)

    out_descs = [None, None]
    for c in range(NCHUNK):
        p = c & 1
        if c + 1 < NCHUNK:
            descs[c + 1] = fire(c + 1)
        for d in descs.pop(c):
            d.wait()
        if c >= 2:
            out_descs[p].wait()

        def sample_body(s, _, p=p, c=c):
            splat = zero_i + s
            scv = plsc.load_gather(scale_v, [splat + c * CHUNK])
            n0 = plsc.load_gather(n0_v, [splat + c * CHUNK])
            te, to = plsc.unpack(trows[p][s, :], format=_IL)
            plsc.store_scatter(rowbuf[p], [splat, iota2], te)
            plsc.store_scatter(rowbuf[p], [splat, iota2 + 1], to)
            acc_e = -n0 * r0e
            acc_o = -n0 * r0o
            rbase = s * SEQ
            for t in range(SEQ):
                e, o = plsc.unpack(tokbuf[p][rbase + t, :], format=_IL)
                acc_e = acc_e + e
                acc_o = acc_o + o
            plsc.store_scatter(rowbuf[p], [splat, iota2 + EMBED], acc_e * scv)
            plsc.store_scatter(rowbuf[p], [splat, iota2 + EMBED + 1],
                               acc_o * scv)
            return 0

        lax.fori_loop(0, CHUNK, sample_body, 0)
        out_descs[p] = pltpu.async_copy(
            rowbuf[p], out_hbm.at[pl.ds(base + c * CHUNK, CHUNK)], sem_o[p])

    out_descs[0].wait()
    out_descs[1].wait()


def kernel(title_ids, text_token_ids, title_table, text_table):
    return _sc_kernel(title_ids.astype(jnp.int32),
                      text_token_ids.astype(jnp.int32),
                      title_table.astype(jnp.bfloat16),
                      text_table.astype(jnp.bfloat16))


# speedup vs baseline: 1.2560x; 1.2560x over previous
import functools

import jax
import jax.numpy as jnp
from jax import lax
from jax.experimental import pallas as pl
from jax.experimental.pallas import tpu as pltpu
from jax.experimental.pallas import tpu_sc as plsc

BATCH = 16384
EMBED = 32
SEQ = 20
SEQP = 24
L = 16

NC = 2
NS = 16
NW = NC * NS
BPW = BATCH // NW
CHUNK = 64
NCHUNK = BPW // CHUNK
TOK_PER_CHUNK = CHUNK * SEQ
IDX_DMA = 128

_mesh = plsc.VectorSubcoreMesh(core_axis_name="c", subcore_axis_name="s")

_IL = plsc.PackFormat.INTERLEAVED


@functools.partial(
    pl.kernel,
    out_type=jax.ShapeDtypeStruct((BATCH, 2 * EMBED), jnp.float32),
    mesh=_mesh,
    compiler_params=pltpu.CompilerParams(needs_layout_passes=False,
                                         use_tc_tiling_on_sc=False),
    scratch_types=(
        [pltpu.VMEM((BPW,), jnp.int32),
         pltpu.VMEM((BPW, SEQP), jnp.int32),
         pltpu.VMEM((BPW,), jnp.float32),
         pltpu.VMEM((BPW,), jnp.float32),
         pltpu.VMEM((1, EMBED), jnp.bfloat16)]
        + [pltpu.VMEM((TOK_PER_CHUNK,), jnp.int32)] * 2
        + [pltpu.VMEM((CHUNK, EMBED), jnp.float32)] * 2
        + [pltpu.VMEM((TOK_PER_CHUNK, EMBED), jnp.bfloat16)] * 2
        + [pltpu.VMEM((CHUNK, 2 * EMBED), jnp.float32)] * 2
        + [pltpu.SemaphoreType.DMA] * 4
    ),
)
def _sc_kernel(title_hbm, tok2d_hbm, ttab_hbm, xtab_hbm, out_hbm,
               tidx_v, tokidx_v, n0_v, scale_v, row0_v,
               flat0, flat1, trows0, trows1, tokbuf0, tokbuf1,
               rowbuf0, rowbuf1, sg0, sg1, so0, so1):
    wid = lax.axis_index("s") * NC + lax.axis_index("c")
    base = wid * BPW
    flat = (flat0, flat1)
    trows = (trows0, trows1)
    tokbuf = (tokbuf0, tokbuf1)
    rowbuf = (rowbuf0, rowbuf1)
    sem_g = (sg0, sg1)
    sem_o = (so0, so1)

    pltpu.sync_copy(title_hbm.at[pl.ds(base, BPW)], tidx_v)
    pltpu.sync_copy(tok2d_hbm.at[pl.ds(base, BPW)], tokidx_v)
    pltpu.sync_copy(xtab_hbm.at[pl.ds(0, 1)], row0_v)

    iota = lax.iota(jnp.int32, L)
    seqv = jnp.full((L,), SEQ, jnp.int32)

    def fire(c):
        p = c & 1

        def build_body(k, _):
            q = c * TOK_PER_CHUNK + k * L + iota
            flat[p][pl.ds(k * L, L)] = plsc.load_gather(
                tokidx_v, [q // seqv, q % seqv])
            return 0

        lax.fori_loop(0, TOK_PER_CHUNK // L, build_body, 0)

        descs = [pltpu.async_copy(
            ttab_hbm.at[tidx_v.at[pl.ds(c * CHUNK, CHUNK)]],
            trows[p], sem_g[p])]
        for j in range(TOK_PER_CHUNK // IDX_DMA):
            descs.append(pltpu.async_copy(
                xtab_hbm.at[flat[p].at[pl.ds(j * IDX_DMA, IDX_DMA)]],
                tokbuf[p].at[pl.ds(j * IDX_DMA, IDX_DMA)], sem_g[p]))
        return descs

    descs = {0: fire(0)}

    one = jnp.ones((L,), jnp.float32)
    zero = jnp.zeros((L,), jnp.float32)
    full = jnp.full((L,), float(SEQ), jnp.float32)

    def count_body(g, _):
        sidx = iota + g * L
        zc = zero
        for t in range(SEQ):
            ids = plsc.load_gather(tokidx_v, [sidx, jnp.full((L,), t,
                                                             jnp.int32)])
            zc = zc + jnp.where(ids == 0, one, zero)
        n0_v[pl.ds(g * L, L)] = zc
        scale_v[pl.ds(g * L, L)] = one / jnp.maximum(full - zc, one)
        return 0

    lax.fori_loop(0, BPW // L, count_body, 0)

    r0e, r0o = plsc.unpack(row0_v[0, :], format=_IL)
    iota2 = iota * 2
    zero_i = jnp.zeros((L,), jnp.int32)

    out_descs = [None, None]
    for c in range(NCHUNK):
        p = c & 1
        if c + 1 < NCHUNK:
            descs[c + 1] = fire(c + 1)
        for d in descs.pop(c):
            d.wait()
        if c >= 2:
            out_descs[p].wait()

        def sample_body(s, _, p=p, c=c):
            splat = zero_i + s
            scv = plsc.load_gather(scale_v, [splat + c * CHUNK])
            n0 = plsc.load_gather(n0_v, [splat + c * CHUNK])
            rowbuf[p][s, pl.ds(0, L)] = trows[p][s, pl.ds(0, L)]
            rowbuf[p][s, pl.ds(L, L)] = trows[p][s, pl.ds(L, L)]
            acc_e = -n0 * r0e
            acc_o = -n0 * r0o
            rbase = s * SEQ
            for t in range(SEQ):
                e, o = plsc.unpack(tokbuf[p][rbase + t, :], format=_IL)
                acc_e = acc_e + e
                acc_o = acc_o + o
            plsc.store_scatter(rowbuf[p], [splat, iota2 + EMBED], acc_e * scv)
            plsc.store_scatter(rowbuf[p], [splat, iota2 + EMBED + 1],
                               acc_o * scv)
            return 0

        lax.fori_loop(0, CHUNK, sample_body, 0)
        out_descs[p] = pltpu.async_copy(
            rowbuf[p], out_hbm.at[pl.ds(base + c * CHUNK, CHUNK)], sem_o[p])

    out_descs[0].wait()
    out_descs[1].wait()


def kernel(title_ids, text_token_ids, title_table, text_table):
    ids_p = jnp.pad(text_token_ids.astype(jnp.int32),
                    ((0, 0), (0, SEQP - SEQ)))
    return _sc_kernel(title_ids.astype(jnp.int32), ids_p,
                      title_table,
                      text_table.astype(jnp.bfloat16))


# speedup vs baseline: 1.3796x; 1.0984x over previous
import functools

import jax
import jax.numpy as jnp
from jax import lax
from jax.experimental import pallas as pl
from jax.experimental.pallas import tpu as pltpu
from jax.experimental.pallas import tpu_sc as plsc

BATCH = 16384
EMBED = 32
SEQ = 20
L = 16

NC = 2
NS = 16
NW = NC * NS
BPW = BATCH // NW
CHUNK = 64
NCHUNK = BPW // CHUNK
TOK_PER_CHUNK = CHUNK * SEQ
IDX_DMA = 128

_mesh = plsc.VectorSubcoreMesh(core_axis_name="c", subcore_axis_name="s")
_params = pltpu.CompilerParams(needs_layout_passes=False,
                               use_tc_tiling_on_sc=False)
_IL = plsc.PackFormat.INTERLEAVED


@functools.partial(
    pl.kernel,
    out_type=jax.ShapeDtypeStruct((BATCH, EMBED), jnp.float32),
    mesh=_mesh,
    compiler_params=_params,
    scratch_types=(
        [pltpu.VMEM((BPW, SEQ), jnp.int32),
         pltpu.VMEM((BPW,), jnp.float32),
         pltpu.VMEM((BPW,), jnp.float32),
         pltpu.VMEM((1, EMBED), jnp.bfloat16)]
        + [pltpu.VMEM((TOK_PER_CHUNK,), jnp.int32)] * 2
        + [pltpu.VMEM((TOK_PER_CHUNK, EMBED), jnp.bfloat16)] * 2
        + [pltpu.VMEM((CHUNK, EMBED), jnp.float32)] * 2
        + [pltpu.SemaphoreType.DMA] * 4
    ),
)
def _text_kernel(tok2d_hbm, xtab_hbm, out_hbm,
                 tokidx_v, n0_v, scale_v, row0_v,
                 flat0, flat1, tokbuf0, tokbuf1,
                 rowbuf0, rowbuf1, sg0, sg1, so0, so1):
    wid = lax.axis_index("s") * NC + lax.axis_index("c")
    base = wid * BPW
    flat = (flat0, flat1)
    tokbuf = (tokbuf0, tokbuf1)
    rowbuf = (rowbuf0, rowbuf1)
    sem_g = (sg0, sg1)
    sem_o = (so0, so1)

    pltpu.sync_copy(tok2d_hbm.at[pl.ds(base, BPW)], tokidx_v)
    pltpu.sync_copy(xtab_hbm.at[pl.ds(0, 1)], row0_v)

    iota = lax.iota(jnp.int32, L)
    seqv = jnp.full((L,), SEQ, jnp.int32)

    def fire(c):
        p = c & 1

        def build_body(k, _):
            q = c * TOK_PER_CHUNK + k * L + iota
            flat[p][pl.ds(k * L, L)] = plsc.load_gather(
                tokidx_v, [q // seqv, q % seqv])
            return 0

        lax.fori_loop(0, TOK_PER_CHUNK // L, build_body, 0)

        descs = []
        for j in range(TOK_PER_CHUNK // IDX_DMA):
            descs.append(pltpu.async_copy(
                xtab_hbm.at[flat[p].at[pl.ds(j * IDX_DMA, IDX_DMA)]],
                tokbuf[p].at[pl.ds(j * IDX_DMA, IDX_DMA)], sem_g[p]))
        return descs

    descs = {0: fire(0)}

    one = jnp.ones((L,), jnp.float32)
    zero = jnp.zeros((L,), jnp.float32)
    full = jnp.full((L,), float(SEQ), jnp.float32)

    def count_body(g, _):
        sidx = iota + g * L
        zc = zero
        for t in range(SEQ):
            ids = plsc.load_gather(tokidx_v, [sidx, jnp.full((L,), t,
                                                             jnp.int32)])
            zc = zc + jnp.where(ids == 0, one, zero)
        n0_v[pl.ds(g * L, L)] = zc
        scale_v[pl.ds(g * L, L)] = one / jnp.maximum(full - zc, one)
        return 0

    lax.fori_loop(0, BPW // L, count_body, 0)

    r0e, r0o = plsc.unpack(row0_v[0, :], format=_IL)
    iota2 = iota * 2
    zero_i = jnp.zeros((L,), jnp.int32)

    out_descs = [None, None]
    for c in range(NCHUNK):
        p = c & 1
        if c + 1 < NCHUNK:
            descs[c + 1] = fire(c + 1)
        for d in descs.pop(c):
            d.wait()
        if c >= 2:
            out_descs[p].wait()

        def sample_body(s, _, p=p, c=c):
            splat = zero_i + s
            scv = plsc.load_gather(scale_v, [splat + c * CHUNK])
            n0 = plsc.load_gather(n0_v, [splat + c * CHUNK])
            acc_e = -n0 * r0e
            acc_o = -n0 * r0o
            rbase = s * SEQ
            for t in range(SEQ):
                e, o = plsc.unpack(tokbuf[p][rbase + t, :], format=_IL)
                acc_e = acc_e + e
                acc_o = acc_o + o
            plsc.store_scatter(rowbuf[p], [splat, iota2], acc_e * scv)
            plsc.store_scatter(rowbuf[p], [splat, iota2 + 1], acc_o * scv)
            return 0

        lax.fori_loop(0, CHUNK, sample_body, 0)
        out_descs[p] = pltpu.async_copy(
            rowbuf[p], out_hbm.at[pl.ds(base + c * CHUNK, CHUNK)], sem_o[p])

    out_descs[0].wait()
    out_descs[1].wait()


@functools.partial(
    pl.kernel,
    out_type=jax.ShapeDtypeStruct((BATCH, EMBED), jnp.float32),
    mesh=_mesh,
    compiler_params=_params,
    scratch_types=[
        pltpu.VMEM((BPW,), jnp.int32),
        pltpu.VMEM((BPW, EMBED), jnp.float32),
        pltpu.SemaphoreType.DMA,
    ],
)
def _title_kernel(title_hbm, ttab_hbm, out_hbm, tidx_v, trows_v, sem):
    wid = lax.axis_index("s") * NC + lax.axis_index("c")
    base = wid * BPW
    pltpu.sync_copy(title_hbm.at[pl.ds(base, BPW)], tidx_v)
    descs = [
        pltpu.async_copy(
            ttab_hbm.at[tidx_v.at[pl.ds(j * IDX_DMA, IDX_DMA)]],
            trows_v.at[pl.ds(j * IDX_DMA, IDX_DMA)], sem)
        for j in range(BPW // IDX_DMA)
    ]
    for d in descs:
        d.wait()
    pltpu.sync_copy(trows_v, out_hbm.at[pl.ds(base, BPW)])


def kernel(title_ids, text_token_ids, title_table, text_table):
    text_half = _text_kernel(text_token_ids.astype(jnp.int32),
                             text_table.astype(jnp.bfloat16))
    title_half = _title_kernel(title_ids.astype(jnp.int32), title_table)
    return jnp.concatenate([title_half, text_half], axis=1)
